# baseline (speedup 1.0000x reference)
import functools

import jax
import jax.numpy as jnp
from jax import lax
from jax.experimental import pallas as pl
from jax.experimental.pallas import tpu as pltpu
from jax.experimental.pallas import tpu_sc as plsc

B = 4
N = 50000
K = 16
CIN = 3
COUT = 16

NW = 32
ROWS = N * K
ROWS_PER_W = ROWS // NW
CHUNK = 5000

NB = 400


def _sc_gather(table, idx_flat):
    mesh = plsc.VectorSubcoreMesh(core_axis_name="c", subcore_axis_name="s")

    @functools.partial(
        pl.kernel,
        out_type=jax.ShapeDtypeStruct((B, ROWS, CIN), jnp.float32),
        mesh=mesh,
        compiler_params=pltpu.CompilerParams(use_tc_tiling_on_sc=False),
        scratch_types=[
            pltpu.VMEM((CHUNK,), jnp.int32),
            pltpu.VMEM((CHUNK, 16), jnp.float32),
            pltpu.SemaphoreType.DMA,
        ],
    )
    def k(table_hbm, idx_hbm, out_hbm, idx_v, rows_v, sem):
        wid = lax.axis_index("s") * 2 + lax.axis_index("c")
        base = wid * ROWS_PER_W

        def body(i, carry):
            off = base + i * CHUNK
            pltpu.sync_copy(idx_hbm.at[pl.ds(off, CHUNK)], idx_v)
            pltpu.async_copy(table_hbm.at[idx_v], rows_v, sem).wait()
            for b in range(B):
                pltpu.sync_copy(
                    rows_v.at[:, pl.ds(4 * b, CIN)],
                    out_hbm.at[b, pl.ds(off, CHUNK), :],
                )
            return carry

        lax.fori_loop(0, ROWS_PER_W // CHUNK, body, 0)

    return k(table, idx_flat)


def _tc_body(g_ref, w_ref, b_ref, o_ref):
    w = w_ref[...]
    bias = b_ref[...]
    for b in range(B):
        g48 = g_ref[b]
        gexp = jnp.concatenate([g48] * COUT, axis=1)
        p = gexp * w
        ob = p.reshape(NB, COUT, K * CIN).sum(axis=-1) + bias
        o_ref[b] = jnp.where(ob > 0, ob, jnp.exp(ob) - 1.0)


def _tc_reduce(gsel, weights_om, bias):
    grid = N // NB
    return pl.pallas_call(
        _tc_body,
        grid=(grid,),
        in_specs=[
            pl.BlockSpec((B, NB, K * CIN), lambda i: (0, i, 0)),
            pl.BlockSpec((NB, 768), lambda i: (i, 0)),
            pl.BlockSpec((NB, COUT), lambda i: (i, 0)),
        ],
        out_specs=pl.BlockSpec((B, NB, COUT), lambda i: (0, i, 0)),
        out_shape=jax.ShapeDtypeStruct((B, N, COUT), jnp.float32),
    )(gsel, weights_om, bias)


def kernel(x, neighbor_id_lstlst, weights, bias):
    xt = jnp.transpose(x, (1, 0, 2))
    xt = jnp.pad(xt, ((0, 0), (0, 0), (0, 1)))
    table = xt.reshape(N, 16)
    idx_flat = neighbor_id_lstlst.astype(jnp.int32).reshape(ROWS)

    gsel = _sc_gather(table, idx_flat)
    g3 = gsel.reshape(B, N, K * CIN)
    wom = jnp.transpose(weights, (0, 2, 1, 3)).reshape(N, K * COUT * CIN)
    return _tc_reduce(g3, wom, bias)


# speedup vs baseline: 3.5939x; 3.5939x over previous
import functools

import jax
import jax.numpy as jnp
from jax import lax
from jax.experimental import pallas as pl
from jax.experimental.pallas import tpu as pltpu
from jax.experimental.pallas import tpu_sc as plsc

B = 4
N = 50000
K = 16
CIN = 3
COUT = 16

NW = 32
ROWS = N * K
ROWS_PER_W = ROWS // NW
CHUNK = 5000

NB = 400


def _sc_gather(table, idx_flat):
    mesh = plsc.VectorSubcoreMesh(core_axis_name="c", subcore_axis_name="s")

    @functools.partial(
        pl.kernel,
        out_type=jax.ShapeDtypeStruct((ROWS, 16), jnp.float32),
        mesh=mesh,
        compiler_params=pltpu.CompilerParams(use_tc_tiling_on_sc=False),
        scratch_types=[
            pltpu.VMEM((CHUNK,), jnp.int32),
            pltpu.VMEM((CHUNK, 16), jnp.float32),
            pltpu.SemaphoreType.DMA,
        ],
    )
    def k(table_hbm, idx_hbm, out_hbm, idx_v, rows_v, sem):
        wid = lax.axis_index("s") * 2 + lax.axis_index("c")
        base = wid * ROWS_PER_W

        def body(i, carry):
            off = base + i * CHUNK
            pltpu.sync_copy(idx_hbm.at[pl.ds(off, CHUNK)], idx_v)
            pltpu.async_copy(table_hbm.at[idx_v], rows_v, sem).wait()
            pltpu.sync_copy(rows_v, out_hbm.at[pl.ds(off, CHUNK)])
            return carry

        lax.fori_loop(0, ROWS_PER_W // CHUNK, body, 0)

    return k(table, idx_flat)


def _tc_body(g_ref, w_ref, b_ref, o_ref):
    g = g_ref[...]
    w = w_ref[...]
    bias = b_ref[...]
    for b in range(B):
        g48 = jnp.concatenate(
            [g[:, 16 * k + 4 * b:16 * k + 4 * b + 3] for k in range(K)],
            axis=1)
        gexp = jnp.concatenate([g48] * COUT, axis=1)
        p = gexp * w
        ob = p.reshape(NB, COUT, K * CIN).sum(axis=-1) + bias
        o_ref[b] = jnp.where(ob > 0, ob, jnp.exp(ob) - 1.0)


def _tc_reduce(g2, weights_om, bias):
    grid = N // NB
    return pl.pallas_call(
        _tc_body,
        grid=(grid,),
        in_specs=[
            pl.BlockSpec((NB, K * 16), lambda i: (i, 0)),
            pl.BlockSpec((NB, 768), lambda i: (i, 0)),
            pl.BlockSpec((NB, COUT), lambda i: (i, 0)),
        ],
        out_specs=pl.BlockSpec((B, NB, COUT), lambda i: (0, i, 0)),
        out_shape=jax.ShapeDtypeStruct((B, N, COUT), jnp.float32),
    )(g2, weights_om, bias)


def kernel(x, neighbor_id_lstlst, weights, bias):
    xt = jnp.transpose(x, (1, 0, 2))
    xt = jnp.pad(xt, ((0, 0), (0, 0), (0, 1)))
    table = xt.reshape(N, 16)
    idx_flat = neighbor_id_lstlst.astype(jnp.int32).reshape(ROWS)

    gathered = _sc_gather(table, idx_flat)
    g2 = gathered.reshape(N, K * 16)
    wom = jnp.transpose(weights, (0, 2, 1, 3)).reshape(N, K * COUT * CIN)
    return _tc_reduce(g2, wom, bias)


# speedup vs baseline: 11.2231x; 3.1229x over previous
import functools

import jax
import jax.numpy as jnp
from jax import lax
from jax.experimental import pallas as pl
from jax.experimental.pallas import tpu as pltpu
from jax.experimental.pallas import tpu_sc as plsc

B = 4
N = 50000
K = 16
CIN = 3
COUT = 16

NW = 32
ROWS = N * K
ROWS_PER_W = ROWS // NW
CHUNK = 5000

NB = 512
GRID = (N + NB - 1) // NB


def _sc_gather(table, idx_flat):
    mesh = plsc.VectorSubcoreMesh(core_axis_name="c", subcore_axis_name="s")

    @functools.partial(
        pl.kernel,
        out_type=jax.ShapeDtypeStruct((ROWS, 16), jnp.float32),
        mesh=mesh,
        compiler_params=pltpu.CompilerParams(use_tc_tiling_on_sc=False),
        scratch_types=[
            pltpu.VMEM((CHUNK,), jnp.int32),
            pltpu.VMEM((CHUNK, 16), jnp.float32),
            pltpu.SemaphoreType.DMA,
        ],
    )
    def k(table_hbm, idx_hbm, out_hbm, idx_v, rows_v, sem):
        wid = lax.axis_index("s") * 2 + lax.axis_index("c")
        base = wid * ROWS_PER_W

        def body(i, carry):
            off = base + i * CHUNK
            pltpu.sync_copy(idx_hbm.at[pl.ds(off, CHUNK)], idx_v)
            pltpu.async_copy(table_hbm.at[idx_v], rows_v, sem).wait()
            pltpu.sync_copy(rows_v, out_hbm.at[pl.ds(off, CHUNK)])
            return carry

        lax.fori_loop(0, ROWS_PER_W // CHUNK, body, 0)

    return k(table, idx_flat)


def _tc_body(g_ref, w_ref, b_ref, o_ref):
    g = g_ref[...]
    gt = jnp.transpose(g, (0, 2, 1))
    w3 = w_ref[...].reshape(COUT, K * CIN, NB)
    bias = b_ref[...]
    for b in range(B):
        g48 = gt[:, 4 * b:4 * b + 3, :].reshape(K * CIN, NB)
        prod = w3 * g48[None, :, :]
        ob = prod.sum(axis=1) + bias
        o_ref[b] = jnp.where(ob > 0, ob, jnp.exp(ob) - 1.0)


def _tc_reduce(g3, wt, bias_t):
    return pl.pallas_call(
        _tc_body,
        grid=(GRID,),
        in_specs=[
            pl.BlockSpec((K, NB, 16), lambda i: (0, i, 0)),
            pl.BlockSpec((768, NB), lambda i: (0, i)),
            pl.BlockSpec((COUT, NB), lambda i: (0, i)),
        ],
        out_specs=pl.BlockSpec((B, COUT, NB), lambda i: (0, 0, i)),
        out_shape=jax.ShapeDtypeStruct((B, COUT, N), jnp.float32),
    )(g3, wt, bias_t)


def kernel(x, neighbor_id_lstlst, weights, bias):
    xt = jnp.transpose(x, (1, 0, 2))
    xt = jnp.pad(xt, ((0, 0), (0, 0), (0, 1)))
    table = xt.reshape(N, 16)
    idx_kmaj = jnp.transpose(neighbor_id_lstlst.astype(jnp.int32)).reshape(ROWS)

    gathered = _sc_gather(table, idx_kmaj)
    g3 = gathered.reshape(K, N, 16)
    wt = jnp.transpose(weights, (2, 1, 3, 0)).reshape(768, N)
    bias_t = jnp.transpose(bias)
    out_t = _tc_reduce(g3, wt, bias_t)
    return jnp.transpose(out_t, (0, 2, 1))


# speedup vs baseline: 15.6144x; 1.3913x over previous
import functools

import jax
import jax.numpy as jnp
from jax import lax
from jax.experimental import pallas as pl
from jax.experimental.pallas import tpu as pltpu
from jax.experimental.pallas import tpu_sc as plsc

B = 4
N = 50000
K = 16
CIN = 3
COUT = 16

NW = 32
ROWS = N * K
ROWS_PER_W = ROWS // NW
CHUNK = 5000

NB = 512
GRID = (N + NB - 1) // NB


def _sc_gather(table, idx_flat):
    mesh = plsc.VectorSubcoreMesh(core_axis_name="c", subcore_axis_name="s")

    @functools.partial(
        pl.kernel,
        out_type=jax.ShapeDtypeStruct((ROWS, 16), jnp.float32),
        mesh=mesh,
        compiler_params=pltpu.CompilerParams(use_tc_tiling_on_sc=False),
        scratch_types=[
            pltpu.VMEM((CHUNK,), jnp.int32),
            pltpu.VMEM((CHUNK, 16), jnp.float32),
            pltpu.SemaphoreType.DMA,
        ],
    )
    def k(table_hbm, idx_hbm, out_hbm, idx_v, rows_v, sem):
        wid = lax.axis_index("s") * 2 + lax.axis_index("c")
        base = wid * ROWS_PER_W

        def body(i, carry):
            off = base + i * CHUNK
            pltpu.sync_copy(idx_hbm.at[pl.ds(off, CHUNK)], idx_v)
            pltpu.async_copy(table_hbm.at[idx_v], rows_v, sem).wait()
            pltpu.sync_copy(rows_v, out_hbm.at[pl.ds(off, CHUNK)])
            return carry

        lax.fori_loop(0, ROWS_PER_W // CHUNK, body, 0)

    return k(table, idx_flat)


def _tc_body(g_ref, w_ref, b_ref, o_ref):
    g = g_ref[...]
    gt = jnp.transpose(g, (1, 0))
    gt3 = gt.reshape(K, 16, NB)
    w3 = w_ref[...].reshape(COUT, K * CIN, NB)
    bias = b_ref[...]
    for b in range(B):
        g48 = gt3[:, 4 * b:4 * b + 3, :].reshape(K * CIN, NB)
        prod = w3 * g48[None, :, :]
        ob = prod.sum(axis=1) + bias
        o_ref[b] = jnp.where(ob > 0, ob, jnp.exp(ob) - 1.0)


def _tc_reduce(g2, wt, bias_t):
    return pl.pallas_call(
        _tc_body,
        grid=(GRID,),
        in_specs=[
            pl.BlockSpec((NB, K * 16), lambda i: (i, 0)),
            pl.BlockSpec((768, NB), lambda i: (0, i)),
            pl.BlockSpec((COUT, NB), lambda i: (0, i)),
        ],
        out_specs=pl.BlockSpec((B, COUT, NB), lambda i: (0, 0, i)),
        out_shape=jax.ShapeDtypeStruct((B, COUT, N), jnp.float32),
    )(g2, wt, bias_t)


def kernel(x, neighbor_id_lstlst, weights, bias):
    xt = jnp.transpose(x, (1, 0, 2))
    xt = jnp.pad(xt, ((0, 0), (0, 0), (0, 1)))
    table = xt.reshape(N, 16)
    idx_flat = neighbor_id_lstlst.astype(jnp.int32).reshape(ROWS)

    gathered = _sc_gather(table, idx_flat)
    g2 = gathered.reshape(N, K * 16)
    wt = jnp.transpose(weights, (2, 1, 3, 0)).reshape(768, N)
    bias_t = jnp.transpose(bias)
    out_t = _tc_reduce(g2, wt, bias_t)
    return jnp.transpose(out_t, (0, 2, 1))


# speedup vs baseline: 15.6635x; 1.0031x over previous
import functools

import jax
import jax.numpy as jnp
from jax import lax
from jax.experimental import pallas as pl
from jax.experimental.pallas import tpu as pltpu
from jax.experimental.pallas import tpu_sc as plsc

B = 4
N = 50000
K = 16
CIN = 3
COUT = 16

NW = 32
ROWS = N * K
ROWS_PER_W = ROWS // NW
CHUNK = 5000

NB = 1024
GRID = (N + NB - 1) // NB


def _sc_gather(table, idx_flat):
    mesh = plsc.VectorSubcoreMesh(core_axis_name="c", subcore_axis_name="s")

    @functools.partial(
        pl.kernel,
        out_type=jax.ShapeDtypeStruct((ROWS, 16), jnp.float32),
        mesh=mesh,
        compiler_params=pltpu.CompilerParams(use_tc_tiling_on_sc=False),
        scratch_types=[
            pltpu.VMEM((CHUNK,), jnp.int32),
            pltpu.VMEM((CHUNK, 16), jnp.float32),
            pltpu.SemaphoreType.DMA,
        ],
    )
    def k(table_hbm, idx_hbm, out_hbm, idx_v, rows_v, sem):
        wid = lax.axis_index("s") * 2 + lax.axis_index("c")
        base = wid * ROWS_PER_W

        def body(i, carry):
            off = base + i * CHUNK
            pltpu.sync_copy(idx_hbm.at[pl.ds(off, CHUNK)], idx_v)
            pltpu.async_copy(table_hbm.at[idx_v], rows_v, sem).wait()
            pltpu.sync_copy(rows_v, out_hbm.at[pl.ds(off, CHUNK)])
            return carry

        lax.fori_loop(0, ROWS_PER_W // CHUNK, body, 0)

    return k(table, idx_flat)


def _tc_body(g_ref, w_ref, b_ref, o_ref):
    g = g_ref[...]
    gt = jnp.transpose(g, (1, 0))
    gt3 = gt.reshape(K, 16, NB)
    w3 = w_ref[...].reshape(COUT, K * CIN, NB)
    bias = b_ref[...]
    for b in range(B):
        g48 = gt3[:, 4 * b:4 * b + 3, :].reshape(K * CIN, NB)
        prod = w3 * g48[None, :, :]
        ob = prod.sum(axis=1) + bias
        o_ref[b] = jnp.where(ob > 0, ob, jnp.exp(ob) - 1.0)


def _tc_reduce(g2, wt, bias_t):
    return pl.pallas_call(
        _tc_body,
        grid=(GRID,),
        in_specs=[
            pl.BlockSpec((NB, K * 16), lambda i: (i, 0)),
            pl.BlockSpec((768, NB), lambda i: (0, i)),
            pl.BlockSpec((COUT, NB), lambda i: (0, i)),
        ],
        out_specs=pl.BlockSpec((B, COUT, NB), lambda i: (0, 0, i)),
        out_shape=jax.ShapeDtypeStruct((B, COUT, N), jnp.float32),
    )(g2, wt, bias_t)


def kernel(x, neighbor_id_lstlst, weights, bias):
    xt = jnp.transpose(x, (1, 0, 2))
    xt = jnp.pad(xt, ((0, 0), (0, 0), (0, 1)))
    table = xt.reshape(N, 16)
    idx_flat = neighbor_id_lstlst.astype(jnp.int32).reshape(ROWS)

    gathered = _sc_gather(table, idx_flat)
    g2 = gathered.reshape(N, K * 16)
    wt = jnp.transpose(weights, (2, 1, 3, 0)).reshape(768, N)
    bias_t = jnp.transpose(bias)
    out_t = _tc_reduce(g2, wt, bias_t)
    return jnp.transpose(out_t, (0, 2, 1))
